# baseline (speedup 1.0000x reference)
import jax
import jax.numpy as jnp
from jax import lax
from jax.experimental import pallas as pl
from jax.experimental.pallas import tpu as pltpu
from jax.experimental.pallas import tpu_sc as plsc

_BATCH = 16384
_D = 32
_NC = 2
_NS = 16
_NW = _NC * _NS
_PER_W = _BATCH // _NW
_CHUNK = 128
_NCHUNK = _PER_W // _CHUNK


def _take16(x, idx):
    return lax.gather(
        x, idx[:, None],
        lax.GatherDimensionNumbers(
            offset_dims=(), collapsed_slice_dims=(0,), start_index_map=(0,)),
        (1,), mode=lax.GatherScatterMode.PROMISE_IN_BOUNDS)


def _lfm_body(users_hbm, items_hbm, ut_hbm, it_hbm, out_hbm,
              idx_u, idx_i, u_rows, i_rows, out_v, sem):
    wid = lax.axis_index("s") * _NC + lax.axis_index("c")
    base_row = wid * _NCHUNK

    for j in range(_NCHUNK):
        pltpu.sync_copy(users_hbm.at[base_row + j], idx_u.at[j])
        pltpu.sync_copy(items_hbm.at[base_row + j], idx_i.at[j])

    copies = []
    for j in range(_NCHUNK):
        copies.append(pltpu.async_copy(ut_hbm.at[idx_u.at[j]], u_rows.at[j], sem))
        copies.append(pltpu.async_copy(it_hbm.at[idx_i.at[j]], i_rows.at[j], sem))
    for c in copies:
        c.wait()

    lane = lax.broadcasted_iota(jnp.int32, (16,), 0)
    perms = [lane ^ m for m in (8, 4, 2, 1)]
    for j in range(_NCHUNK):
        def group(g, _, j=j):
            acc = jnp.zeros((16,), jnp.float32)
            for t in range(16):
                r = g * 16 + t
                u0 = u_rows[j, r, pl.ds(0, 16)]
                u1 = u_rows[j, r, pl.ds(16, 16)]
                v0 = i_rows[j, r, pl.ds(0, 16)]
                v1 = i_rows[j, r, pl.ds(16, 16)]
                s = u0 * v0 + u1 * v1
                for p in perms:
                    s = s + _take16(s, p)
                acc = jnp.where(lane == t, s, acc)
            out_v[pl.ds(j * _CHUNK + g * 16, 16)] = acc
            return 0
        lax.fori_loop(0, _CHUNK // 16, group, 0)

    pltpu.sync_copy(out_v, out_hbm.at[pl.ds(wid * _PER_W, _PER_W)])


def kernel(users, items, user_table, item_table):
    users32 = users.astype(jnp.int32).reshape(_NW * _NCHUNK, _CHUNK)
    items32 = items.astype(jnp.int32).reshape(_NW * _NCHUNK, _CHUNK)
    mesh = plsc.VectorSubcoreMesh(core_axis_name="c", subcore_axis_name="s")
    run = pl.kernel(
        _lfm_body,
        mesh=mesh,
        compiler_params=pltpu.CompilerParams(use_tc_tiling_on_sc=False),
        out_type=jax.ShapeDtypeStruct((_BATCH,), jnp.float32),
        scratch_types=[
            pltpu.VMEM((_NCHUNK, _CHUNK), jnp.int32),
            pltpu.VMEM((_NCHUNK, _CHUNK), jnp.int32),
            pltpu.VMEM((_NCHUNK, _CHUNK, _D), jnp.float32),
            pltpu.VMEM((_NCHUNK, _CHUNK, _D), jnp.float32),
            pltpu.VMEM((_PER_W,), jnp.float32),
            pltpu.SemaphoreType.DMA,
        ],
    )
    return run(users32, items32, user_table, item_table)


# speedup vs baseline: 3.6555x; 3.6555x over previous
import jax
import jax.numpy as jnp
from jax import lax
from jax.experimental import pallas as pl
from jax.experimental.pallas import tpu as pltpu
from jax.experimental.pallas import tpu_sc as plsc

_BATCH = 16384
_D = 32
_NC = 2
_NS = 16
_NW = _NC * _NS
_PER_W = _BATCH // _NW
_RING = 4


def _take16(x, idx):
    return lax.gather(
        x, idx[:, None],
        lax.GatherDimensionNumbers(
            offset_dims=(), collapsed_slice_dims=(0,), start_index_map=(0,)),
        (1,), mode=lax.GatherScatterMode.PROMISE_IN_BOUNDS)


def _lfm_body(users_hbm, items_hbm, ut_hbm, it_hbm, out_hbm,
              su, si, iv, blk_u, blk_i, u_flat, i_flat, out_v, sems):
    wid = lax.axis_index("s") * _NC + lax.axis_index("c")
    base = wid * _PER_W

    pltpu.sync_copy(users_hbm.at[pl.ds(base, _PER_W)], su.at[pl.ds(0, _PER_W)])
    pltpu.sync_copy(items_hbm.at[pl.ds(base, _PER_W)], si.at[pl.ds(0, _PER_W)])
    del iv

    def fire(k, slot):
        u = su[pl.ds(k, 16)][0]
        it = si[pl.ds(k, 16)][0]
        cu = (u >> 7) * 128
        ci = (it >> 7) * 128
        pltpu.async_copy(ut_hbm.at[:, pl.ds(cu, 128)], blk_u.at[slot], sems.at[slot])
        pltpu.async_copy(it_hbm.at[:, pl.ds(ci, 128)], blk_i.at[slot], sems.at[slot])

    def drain(k, slot):
        pltpu.make_async_copy(
            ut_hbm.at[:, pl.ds(0, 128)], blk_u.at[slot], sems.at[slot]).wait()
        pltpu.make_async_copy(
            it_hbm.at[:, pl.ds(0, 128)], blk_i.at[slot], sems.at[slot]).wait()

    lane16 = lax.broadcasted_iota(jnp.int32, (16,), 0)

    def extract(k, slot):
        wu = jnp.full((16,), su[pl.ds(k, 16)][0] & 127, jnp.int32)
        wi = jnp.full((16,), si[pl.ds(k, 16)][0] & 127, jnp.int32)
        slotv = jnp.full((16,), slot, jnp.int32)
        for h in range(2):
            fidx = lane16 + (h * 16)
            u_col = plsc.load_gather(blk_u, [slotv, fidx, wu])
            i_col = plsc.load_gather(blk_i, [slotv, fidx, wi])
            u_flat[pl.ds(k * _D + h * 16, 16)] = u_col
            i_flat[pl.ds(k * _D + h * 16, 16)] = i_col

    for k in range(_RING):
        fire(k, k)

    def step(k, _):
        slot = lax.rem(k, _RING)
        drain(k, slot)
        extract(k, slot)
        fire(k + _RING, slot)
        return 0
    lax.fori_loop(0, _PER_W - _RING, step, 0)

    def tail(k, _):
        slot = lax.rem(k, _RING)
        drain(k, slot)
        extract(k, slot)
        return 0
    lax.fori_loop(_PER_W - _RING, _PER_W, tail, 0)

    lane = lax.broadcasted_iota(jnp.int32, (16,), 0)
    perms = [lane ^ m for m in (8, 4, 2, 1)]

    def group(g, _):
        acc = jnp.zeros((16,), jnp.float32)
        for t in range(16):
            r = g * 16 + t
            u0 = u_flat[pl.ds(r * _D, 16)]
            u1 = u_flat[pl.ds(r * _D + 16, 16)]
            v0 = i_flat[pl.ds(r * _D, 16)]
            v1 = i_flat[pl.ds(r * _D + 16, 16)]
            s = u0 * v0 + u1 * v1
            for p in perms:
                s = s + _take16(s, p)
            acc = jnp.where(lane == t, s, acc)
        out_v[pl.ds(g * 16, 16)] = acc
        return 0
    lax.fori_loop(0, _PER_W // 16, group, 0)

    pltpu.sync_copy(out_v, out_hbm.at[pl.ds(base, _PER_W)])


def kernel(users, items, user_table, item_table):
    users32 = users.astype(jnp.int32)
    items32 = items.astype(jnp.int32)
    ut_t = user_table.T
    it_t = item_table.T
    mesh = plsc.VectorSubcoreMesh(core_axis_name="c", subcore_axis_name="s")
    run = pl.kernel(
        _lfm_body,
        mesh=mesh,
        compiler_params=pltpu.CompilerParams(
            use_tc_tiling_on_sc=True, needs_layout_passes=False),
        out_type=jax.ShapeDtypeStruct((_BATCH,), jnp.float32),
        scratch_types=[
            pltpu.VMEM((_PER_W + 16,), jnp.int32),
            pltpu.VMEM((_PER_W + 16,), jnp.int32),
            pltpu.VMEM((8,), jnp.int32),
            pltpu.VMEM((_RING, _D, 128), jnp.float32),
            pltpu.VMEM((_RING, _D, 128), jnp.float32),
            pltpu.VMEM((_PER_W * _D,), jnp.float32),
            pltpu.VMEM((_PER_W * _D,), jnp.float32),
            pltpu.VMEM((_PER_W,), jnp.float32),
            pltpu.SemaphoreType.DMA((_RING,)),
        ],
    )
    return run(users32, items32, ut_t, it_t)


# speedup vs baseline: 4.2949x; 1.1749x over previous
import jax
import jax.numpy as jnp
from jax import lax
from jax.experimental import pallas as pl
from jax.experimental.pallas import tpu as pltpu
from jax.experimental.pallas import tpu_sc as plsc

_BATCH = 16384
_D = 32
_NC = 2
_NS = 16
_NW = _NC * _NS
_PER_W = _BATCH // _NW
_RING = 8


def _take16(x, idx):
    return lax.gather(
        x, idx[:, None],
        lax.GatherDimensionNumbers(
            offset_dims=(), collapsed_slice_dims=(0,), start_index_map=(0,)),
        (1,), mode=lax.GatherScatterMode.PROMISE_IN_BOUNDS)


def _lfm_body(users_hbm, items_hbm, ut_hbm, it_hbm, out_hbm,
              su, si, iv, blk_u, blk_i, u_flat, i_flat, out_v, sems):
    wid = lax.axis_index("s") * _NC + lax.axis_index("c")
    base = wid * _PER_W

    pltpu.sync_copy(users_hbm.at[pl.ds(base, _PER_W)], su.at[pl.ds(0, _PER_W)])
    pltpu.sync_copy(items_hbm.at[pl.ds(base, _PER_W)], si.at[pl.ds(0, _PER_W)])
    del iv

    def fire(k, slot):
        u = su[pl.ds(k, 16)][0]
        it = si[pl.ds(k, 16)][0]
        cu = (u >> 7) * 128
        ci = (it >> 7) * 128
        pltpu.async_copy(ut_hbm.at[:, pl.ds(cu, 128)], blk_u.at[slot], sems.at[slot])
        pltpu.async_copy(it_hbm.at[:, pl.ds(ci, 128)], blk_i.at[slot], sems.at[slot])

    def drain(k, slot):
        pltpu.make_async_copy(
            ut_hbm.at[:, pl.ds(0, 128)], blk_u.at[slot], sems.at[slot]).wait()
        pltpu.make_async_copy(
            it_hbm.at[:, pl.ds(0, 128)], blk_i.at[slot], sems.at[slot]).wait()

    lane16 = lax.broadcasted_iota(jnp.int32, (16,), 0)

    def extract(k, slot):
        wu = jnp.full((16,), su[pl.ds(k, 16)][0] & 127, jnp.int32)
        wi = jnp.full((16,), si[pl.ds(k, 16)][0] & 127, jnp.int32)
        slotv = jnp.full((16,), slot, jnp.int32)
        for h in range(2):
            fidx = lane16 + (h * 16)
            u_col = plsc.load_gather(blk_u, [slotv, fidx, wu])
            i_col = plsc.load_gather(blk_i, [slotv, fidx, wi])
            u_flat[pl.ds(k * _D + h * 16, 16)] = u_col
            i_flat[pl.ds(k * _D + h * 16, 16)] = i_col

    for k in range(_RING):
        fire(k, k)

    def step(k, _):
        slot = lax.rem(k, _RING)
        drain(k, slot)
        extract(k, slot)
        fire(k + _RING, slot)
        return 0
    lax.fori_loop(0, _PER_W - _RING, step, 0)

    def tail(k, _):
        slot = lax.rem(k, _RING)
        drain(k, slot)
        extract(k, slot)
        return 0
    lax.fori_loop(_PER_W - _RING, _PER_W, tail, 0)

    lane = lax.broadcasted_iota(jnp.int32, (16,), 0)
    perms = [lane ^ m for m in (8, 4, 2, 1)]

    def group(g, _):
        acc = jnp.zeros((16,), jnp.float32)
        for t in range(16):
            r = g * 16 + t
            u0 = u_flat[pl.ds(r * _D, 16)]
            u1 = u_flat[pl.ds(r * _D + 16, 16)]
            v0 = i_flat[pl.ds(r * _D, 16)]
            v1 = i_flat[pl.ds(r * _D + 16, 16)]
            s = u0 * v0 + u1 * v1
            for p in perms:
                s = s + _take16(s, p)
            acc = jnp.where(lane == t, s, acc)
        out_v[pl.ds(g * 16, 16)] = acc
        return 0
    lax.fori_loop(0, _PER_W // 16, group, 0)

    pltpu.sync_copy(out_v, out_hbm.at[pl.ds(base, _PER_W)])


def kernel(users, items, user_table, item_table):
    users32 = users.astype(jnp.int32)
    items32 = items.astype(jnp.int32)
    ut_t = user_table.T
    it_t = item_table.T
    mesh = plsc.VectorSubcoreMesh(core_axis_name="c", subcore_axis_name="s")
    run = pl.kernel(
        _lfm_body,
        mesh=mesh,
        compiler_params=pltpu.CompilerParams(
            use_tc_tiling_on_sc=True, needs_layout_passes=False),
        out_type=jax.ShapeDtypeStruct((_BATCH,), jnp.float32),
        scratch_types=[
            pltpu.VMEM((_PER_W + 16,), jnp.int32),
            pltpu.VMEM((_PER_W + 16,), jnp.int32),
            pltpu.VMEM((8,), jnp.int32),
            pltpu.VMEM((_RING, _D, 128), jnp.float32),
            pltpu.VMEM((_RING, _D, 128), jnp.float32),
            pltpu.VMEM((_PER_W * _D,), jnp.float32),
            pltpu.VMEM((_PER_W * _D,), jnp.float32),
            pltpu.VMEM((_PER_W,), jnp.float32),
            pltpu.SemaphoreType.DMA((_RING,)),
        ],
    )
    return run(users32, items32, ut_t, it_t)


# speedup vs baseline: 4.4063x; 1.0259x over previous
import jax
import jax.numpy as jnp
from jax import lax
from jax.experimental import pallas as pl
from jax.experimental.pallas import tpu as pltpu
from jax.experimental.pallas import tpu_sc as plsc

_BATCH = 16384
_D = 32
_NC = 2
_NS = 16
_NW = _NC * _NS
_PER_W = _BATCH // _NW
_RING = 14


def _take16(x, idx):
    return lax.gather(
        x, idx[:, None],
        lax.GatherDimensionNumbers(
            offset_dims=(), collapsed_slice_dims=(0,), start_index_map=(0,)),
        (1,), mode=lax.GatherScatterMode.PROMISE_IN_BOUNDS)


def _lfm_body(users_hbm, items_hbm, ut_hbm, it_hbm, out_hbm,
              su, si, blk_u, blk_i, out_v, sems):
    wid = lax.axis_index("s") * _NC + lax.axis_index("c")
    base = wid * _PER_W

    pltpu.sync_copy(users_hbm.at[pl.ds(base, _PER_W)], su.at[pl.ds(0, _PER_W)])
    pltpu.sync_copy(items_hbm.at[pl.ds(base, _PER_W)], si.at[pl.ds(0, _PER_W)])

    lane = lax.broadcasted_iota(jnp.int32, (16,), 0)
    perms = [lane ^ m for m in (8, 4, 2, 1)]

    def fire(k, slot):
        u = su[pl.ds(k, 16)][0]
        it = si[pl.ds(k, 16)][0]
        cu = (u >> 7) * 128
        ci = (it >> 7) * 128
        pltpu.async_copy(ut_hbm.at[:, pl.ds(cu, 128)], blk_u.at[slot], sems.at[slot])
        pltpu.async_copy(it_hbm.at[:, pl.ds(ci, 128)], blk_i.at[slot], sems.at[slot])

    def drain(slot):
        pltpu.make_async_copy(
            ut_hbm.at[:, pl.ds(0, 128)], blk_u.at[slot], sems.at[slot]).wait()
        pltpu.make_async_copy(
            it_hbm.at[:, pl.ds(0, 128)], blk_i.at[slot], sems.at[slot]).wait()

    def compute(k, slot, acc):
        wu = jnp.full((16,), su[pl.ds(k, 16)][0] & 127, jnp.int32)
        wi = jnp.full((16,), si[pl.ds(k, 16)][0] & 127, jnp.int32)
        slotv = jnp.full((16,), slot, jnp.int32)
        u0 = plsc.load_gather(blk_u, [slotv, lane, wu])
        u1 = plsc.load_gather(blk_u, [slotv, lane + 16, wu])
        v0 = plsc.load_gather(blk_i, [slotv, lane, wi])
        v1 = plsc.load_gather(blk_i, [slotv, lane + 16, wi])
        s = u0 * v0 + u1 * v1
        for p in perms:
            s = s + _take16(s, p)
        acc = jnp.where(lane == (k & 15), s, acc)

        @pl.when((k & 15) == 15)
        def _():
            out_v[pl.ds(k & ~15, 16)] = acc
        return acc

    for k in range(_RING):
        fire(k, k)

    def step(k, acc):
        slot = lax.rem(k, _RING)
        drain(slot)
        acc = compute(k, slot, acc)
        fire(k + _RING, slot)
        return acc
    acc = lax.fori_loop(0, _PER_W - _RING, step, jnp.zeros((16,), jnp.float32))

    def tail(k, acc):
        slot = lax.rem(k, _RING)
        drain(slot)
        return compute(k, slot, acc)
    lax.fori_loop(_PER_W - _RING, _PER_W, tail, acc)

    pltpu.sync_copy(out_v, out_hbm.at[pl.ds(base, _PER_W)])


def kernel(users, items, user_table, item_table):
    users32 = users.astype(jnp.int32)
    items32 = items.astype(jnp.int32)
    ut_t = user_table.T
    it_t = item_table.T
    mesh = plsc.VectorSubcoreMesh(core_axis_name="c", subcore_axis_name="s")
    run = pl.kernel(
        _lfm_body,
        mesh=mesh,
        compiler_params=pltpu.CompilerParams(
            use_tc_tiling_on_sc=True, needs_layout_passes=False),
        out_type=jax.ShapeDtypeStruct((_BATCH,), jnp.float32),
        scratch_types=[
            pltpu.VMEM((_PER_W + 16,), jnp.int32),
            pltpu.VMEM((_PER_W + 16,), jnp.int32),
            pltpu.VMEM((_RING, _D, 128), jnp.float32),
            pltpu.VMEM((_RING, _D, 128), jnp.float32),
            pltpu.VMEM((_PER_W,), jnp.float32),
            pltpu.SemaphoreType.DMA((_RING,)),
        ],
    )
    return run(users32, items32, ut_t, it_t)
